# baseline (speedup 1.0000x reference)
import jax
import jax.numpy as jnp
from jax import lax
from jax.experimental import pallas as pl
from jax.experimental.pallas import tpu as pltpu

B, C, H, W = 64, 768, 32, 32
HW = H * W
HALF = C // 2

R = HALF
N_CHUNKS = B * C // R
NBUF = 12


def _in_copy(fm_hbm, in_bufs, in_sems, chunk, slot):
    return pltpu.make_async_copy(
        fm_hbm.at[pl.ds(chunk * R, R)], in_bufs.at[slot], in_sems.at[slot]
    )


def _out_copy(out_hbm, out_bufs, out_sems, chunk, slot):
    return pltpu.make_async_copy(
        out_bufs.at[slot], out_hbm.at[pl.ds(chunk * R, R)], out_sems.at[slot]
    )


def _body(emb_ref, fm_hbm, out_hbm, in_bufs, out_bufs, pos_ref, in_sems, out_sems):
    g = pl.program_id(0)
    s = lax.rem(g, NBUF)
    parity = lax.rem(g, 2)

    @pl.when(g == 0)
    def _prologue():
        e = emb_ref[...]
        ii = lax.broadcasted_iota(jnp.int32, (H, HW), 0)
        jj = lax.broadcasted_iota(jnp.int32, (H, HW), 1)
        sel_row = ((jj // W) == ii).astype(jnp.float32)
        sel_col = ((jj % W) == ii).astype(jnp.float32)
        pos_ref[0] = lax.dot(e[:HALF], sel_row, preferred_element_type=jnp.float32)
        pos_ref[1] = lax.dot(e[HALF:], sel_col, preferred_element_type=jnp.float32)
        for k in range(NBUF):
            _in_copy(fm_hbm, in_bufs, in_sems, k, k).start()

    _in_copy(fm_hbm, in_bufs, in_sems, g, s).wait()

    @pl.when(g >= NBUF)
    def _reclaim_out():
        _out_copy(out_hbm, out_bufs, out_sems, g - NBUF, s).wait()

    out_bufs[s] = in_bufs[s] + pos_ref[parity]

    _out_copy(out_hbm, out_bufs, out_sems, g, s).start()

    @pl.when(g + NBUF < N_CHUNKS)
    def _next_in():
        _in_copy(fm_hbm, in_bufs, in_sems, g + NBUF, s).start()

    @pl.when(g == N_CHUNKS - 1)
    def _epilogue():
        for k in range(NBUF):
            chunk = N_CHUNKS - NBUF + k
            _out_copy(out_hbm, out_bufs, out_sems, chunk, lax.rem(chunk, NBUF)).wait()


def kernel(feature_map, row_embed, col_embed):
    emb = jnp.concatenate([row_embed.T, col_embed.T], axis=0)
    fm2 = feature_map.reshape(B * C, HW)

    out = pl.pallas_call(
        _body,
        grid=(N_CHUNKS,),
        in_specs=[
            pl.BlockSpec((C, H), lambda g: (0, 0)),
            pl.BlockSpec(memory_space=pltpu.MemorySpace.HBM),
        ],
        out_specs=pl.BlockSpec(memory_space=pltpu.MemorySpace.HBM),
        out_shape=jax.ShapeDtypeStruct((B * C, HW), jnp.float32),
        scratch_shapes=[
            pltpu.VMEM((NBUF, R, HW), jnp.float32),
            pltpu.VMEM((NBUF, R, HW), jnp.float32),
            pltpu.VMEM((2, R, HW), jnp.float32),
            pltpu.SemaphoreType.DMA((NBUF,)),
            pltpu.SemaphoreType.DMA((NBUF,)),
        ],
    )(emb, fm2)
    return out.reshape(B, C, H, W)


# speedup vs baseline: 2.2298x; 2.2298x over previous
import jax
import jax.numpy as jnp
from jax import lax
from jax.experimental import pallas as pl
from jax.experimental.pallas import tpu as pltpu

B, C, H, W = 64, 768, 32, 32
HW = H * W
HALF = C // 2

B_BLK = 4


def _body(emb_ref, fm_ref, out_ref, pos_ref):
    i = pl.program_id(0)

    @pl.when(i == 0)
    def _build_pos():
        e = emb_ref[...]
        ii = lax.broadcasted_iota(jnp.int32, (H, HW), 0)
        jj = lax.broadcasted_iota(jnp.int32, (H, HW), 1)
        sel_row = ((jj // W) == ii).astype(jnp.float32)
        sel_col = ((jj % W) == ii).astype(jnp.float32)
        pos_ref[:HALF] = lax.dot(e[:HALF], sel_row, preferred_element_type=jnp.float32)
        pos_ref[HALF:] = lax.dot(e[HALF:], sel_col, preferred_element_type=jnp.float32)

    out_ref[...] = fm_ref[...] + pos_ref[...][None]


def kernel(feature_map, row_embed, col_embed):
    emb = jnp.concatenate([row_embed.T, col_embed.T], axis=0)
    fm3 = feature_map.reshape(B, C, HW)

    out = pl.pallas_call(
        _body,
        grid=(B // B_BLK,),
        in_specs=[
            pl.BlockSpec((C, H), lambda i: (0, 0)),
            pl.BlockSpec((B_BLK, C, HW), lambda i: (i, 0, 0)),
        ],
        out_specs=pl.BlockSpec((B_BLK, C, HW), lambda i: (i, 0, 0)),
        out_shape=jax.ShapeDtypeStruct((B, C, HW), jnp.float32),
        scratch_shapes=[pltpu.VMEM((C, HW), jnp.float32)],
    )(emb, fm3)
    return out.reshape(B, C, H, W)
